# Initial kernel scaffold; baseline (speedup 1.0000x reference)
import jax
import jax.numpy as jnp
from jax.experimental import pallas as pl


def kernel(pooled_repr, domain_hint, W_g, domain_bias):
    raise NotImplementedError("write your pallas kernel here")


# speedup vs baseline: 6.5395x; 6.5395x over previous
import jax
import jax.numpy as jnp
from jax.experimental import pallas as pl
from jax.experimental.pallas import tpu as pltpu

_D_MODEL = 4096
_N_EXPERTS = 64
_TOP_K = 8
_BM = 512


def _gating_block(hint_ref, x_ref, w_ref, b_ref, o_ref):
    logits = jax.lax.dot_general(
        x_ref[...], w_ref[...],
        dimension_numbers=(((1,), (0,)), ((), ())),
        preferred_element_type=jnp.float32,
    )
    logits = logits + b_ref[0]

    masked = logits
    rowmax = None
    thresh = None
    for _ in range(_TOP_K):
        thresh = jnp.max(masked, axis=1, keepdims=True)
        if rowmax is None:
            rowmax = thresh
        masked = jnp.where(masked >= thresh, -jnp.inf, masked)

    sel = logits >= thresh
    e = jnp.where(sel, jnp.exp(logits - rowmax), 0.0)
    o_ref[...] = e / jnp.sum(e, axis=1, keepdims=True)


def kernel(pooled_repr, domain_hint, W_g, domain_bias):
    n_tokens, d_model = pooled_repr.shape
    n_experts = W_g.shape[1]
    hint = jnp.asarray(domain_hint, dtype=jnp.int32).reshape((1,))
    bias3 = domain_bias.reshape(domain_bias.shape[0], 1, n_experts)

    grid = (n_tokens // _BM,)
    return pl.pallas_call(
        _gating_block,
        grid_spec=pltpu.PrefetchScalarGridSpec(
            num_scalar_prefetch=1,
            grid=grid,
            in_specs=[
                pl.BlockSpec((_BM, d_model), lambda i, h: (i, 0)),
                pl.BlockSpec((d_model, n_experts), lambda i, h: (0, 0)),
                pl.BlockSpec((1, 1, n_experts), lambda i, h: (h[0], 0, 0)),
            ],
            out_specs=pl.BlockSpec((_BM, n_experts), lambda i, h: (i, 0)),
        ),
        out_shape=jax.ShapeDtypeStruct((n_tokens, n_experts), jnp.float32),
    )(hint, pooled_repr, W_g, bias3)


# speedup vs baseline: 6.5416x; 1.0003x over previous
import jax
import jax.numpy as jnp
from jax.experimental import pallas as pl
from jax.experimental.pallas import tpu as pltpu

_D_MODEL = 4096
_N_EXPERTS = 64
_TOP_K = 8
_BM = 512


def _gating_block(hint_ref, x_ref, w_ref, b_ref, o_ref):
    logits = jax.lax.dot_general(
        x_ref[...], w_ref[...],
        dimension_numbers=(((1,), (0,)), ((), ())),
        preferred_element_type=jnp.float32,
    )
    logits = logits + b_ref[0]

    masked = logits
    rowmax = None
    thresh = None
    for _ in range(_TOP_K):
        thresh = jnp.max(masked, axis=1, keepdims=True)
        if rowmax is None:
            rowmax = thresh
        masked = jnp.where(masked >= thresh, -jnp.inf, masked)

    sel = logits >= thresh
    e = jnp.where(sel, jnp.exp(logits - rowmax), 0.0)
    o_ref[...] = e / jnp.sum(e, axis=1, keepdims=True)


def kernel(pooled_repr, domain_hint, W_g, domain_bias):
    n_tokens, d_model = pooled_repr.shape
    n_experts = W_g.shape[1]
    hint = jnp.asarray(domain_hint, dtype=jnp.int32).reshape((1,))
    bias3 = domain_bias.reshape(domain_bias.shape[0], 1, n_experts)

    grid = (n_tokens // _BM,)
    return pl.pallas_call(
        _gating_block,
        grid_spec=pltpu.PrefetchScalarGridSpec(
            num_scalar_prefetch=1,
            grid=grid,
            in_specs=[
                pl.BlockSpec((_BM, d_model), lambda i, h: (i, 0)),
                pl.BlockSpec((d_model, n_experts), lambda i, h: (0, 0)),
                pl.BlockSpec((1, 1, n_experts), lambda i, h: (h[0], 0, 0)),
            ],
            out_specs=pl.BlockSpec((_BM, n_experts), lambda i, h: (i, 0)),
        ),
        out_shape=jax.ShapeDtypeStruct((n_tokens, n_experts), jnp.float32),
        compiler_params=pltpu.CompilerParams(
            dimension_semantics=("parallel",),
        ),
    )(hint, pooled_repr, W_g, bias3)


# speedup vs baseline: 7.1274x; 1.0896x over previous
import jax
import jax.numpy as jnp
from jax.experimental import pallas as pl
from jax.experimental.pallas import tpu as pltpu

_D_MODEL = 4096
_N_EXPERTS = 64
_TOP_K = 8
_BM = 1024


def _gating_block(hint_ref, x_ref, w_ref, b_ref, o_ref):
    logits = jax.lax.dot_general(
        x_ref[...], w_ref[...],
        dimension_numbers=(((1,), (0,)), ((), ())),
        preferred_element_type=jnp.float32,
    )
    logits = logits + b_ref[0]

    masked = logits
    rowmax = None
    thresh = None
    for _ in range(_TOP_K):
        thresh = jnp.max(masked, axis=1, keepdims=True)
        if rowmax is None:
            rowmax = thresh
        masked = jnp.where(masked >= thresh, -jnp.inf, masked)

    sel = logits >= thresh
    e = jnp.where(sel, jnp.exp(logits - rowmax), 0.0)
    o_ref[...] = e / jnp.sum(e, axis=1, keepdims=True)


def kernel(pooled_repr, domain_hint, W_g, domain_bias):
    n_tokens, d_model = pooled_repr.shape
    n_experts = W_g.shape[1]
    hint = jnp.asarray(domain_hint, dtype=jnp.int32).reshape((1,))
    bias3 = domain_bias.reshape(domain_bias.shape[0], 1, n_experts)

    grid = (n_tokens // _BM,)
    return pl.pallas_call(
        _gating_block,
        grid_spec=pltpu.PrefetchScalarGridSpec(
            num_scalar_prefetch=1,
            grid=grid,
            in_specs=[
                pl.BlockSpec((_BM, d_model), lambda i, h: (i, 0)),
                pl.BlockSpec((d_model, n_experts), lambda i, h: (0, 0)),
                pl.BlockSpec((1, 1, n_experts), lambda i, h: (h[0], 0, 0)),
            ],
            out_specs=pl.BlockSpec((_BM, n_experts), lambda i, h: (i, 0)),
        ),
        out_shape=jax.ShapeDtypeStruct((n_tokens, n_experts), jnp.float32),
        compiler_params=pltpu.CompilerParams(
            dimension_semantics=("parallel",),
        ),
    )(hint, pooled_repr, W_g, bias3)
